# Initial kernel scaffold; baseline (speedup 1.0000x reference)
import jax
import jax.numpy as jnp
from jax.experimental import pallas as pl


def kernel(x, edge_index, Wc0, bc0, m0w1, m0b1, m0w2, m0b2, Wc1, bc1, m1w1, m1b1, m1w2, m1b2, Wc2, bc2, m2w1, m2b1, m2w2, m2b2, fc1_w, fc1_b, fc2_w, fc2_b):
    raise NotImplementedError("write your pallas kernel here")


# speedup vs baseline: 17.5811x; 17.5811x over previous
import functools

import jax
import jax.numpy as jnp
from jax import lax
from jax.experimental import pallas as pl
from jax.experimental.pallas import tpu as pltpu
from jax.experimental.pallas import tpu_sc as plsc

N = 10000
D = 128
E = 320000
H2 = 64

NC = 2
NS = 16
NW = NC * NS

RW = 80
EW = RW * 128
E_PAD = NW * EW
ER = E_PAD // 128
NP = 10112
RPT = NP // NS

BLK = 400

_mesh = plsc.VectorSubcoreMesh(core_axis_name="c", subcore_axis_name="s")


DW = 16

@functools.partial(
    pl.kernel,
    out_type=jax.ShapeDtypeStruct((NC, NP, DW), jnp.float32),
    mesh=_mesh,
    scratch_types=[
        pltpu.VMEM((RW, 128), jnp.int32),
        pltpu.VMEM((128, DW), jnp.float32),
        pltpu.VMEM((16, DW), jnp.float32),
        pltpu.VMEM_SHARED((NP, DW), jnp.float32),
    ],
)
def _deg_call(dst_hbm, out_hbm, didx_v, ones_v, zbuf_v, acc_sh):
    cid = lax.axis_index("c")
    sid = lax.axis_index("s")
    wid = cid * NS + sid

    one16 = jnp.ones((DW,), jnp.float32)
    z16 = jnp.zeros((DW,), jnp.float32)
    for i in range(16):
        zbuf_v[i, pl.ds(0, DW)] = z16
    for i in range(128):
        ones_v[i, pl.ds(0, DW)] = one16

    base = sid * RPT

    def zacc(k, carry):
        pltpu.sync_copy(zbuf_v, acc_sh.at[pl.ds(base + k * 16, 16)])
        return carry

    lax.fori_loop(0, RPT // 16, zacc, 0)
    pltpu.sync_copy(zbuf_v.at[pl.ds(0, RPT - (RPT // 16) * 16)],
                    acc_sh.at[pl.ds(base + (RPT // 16) * 16,
                                    RPT - (RPT // 16) * 16)])
    pltpu.sync_copy(dst_hbm.at[pl.ds(wid * RW, RW)], didx_v)
    plsc.subcore_barrier()

    def body(j, carry):
        pltpu.sync_copy(ones_v, acc_sh.at[didx_v.at[j]], add=True)
        return carry

    lax.fori_loop(0, RW, body, 0)
    plsc.subcore_barrier()
    pltpu.sync_copy(acc_sh.at[pl.ds(base, RPT)],
                    out_hbm.at[cid, pl.ds(base, RPT)])


@functools.partial(
    pl.kernel,
    out_type=jax.ShapeDtypeStruct((NC, NP, D), jnp.float32),
    mesh=_mesh,
    scratch_types=[
        pltpu.VMEM((RW, 128), jnp.int32),
        pltpu.VMEM((RW, 128), jnp.int32),
        pltpu.VMEM((128, D), jnp.float32),
        pltpu.VMEM((16, D), jnp.float32),
        pltpu.VMEM_SHARED((NP, D), jnp.float32),
        pltpu.SemaphoreType.DMA,
    ],
)
def _scat_call(g_hbm, src_hbm, dst_hbm, out_hbm,
               sidx_v, didx_v, rows_v, zbuf_v, acc_sh, sem):
    cid = lax.axis_index("c")
    sid = lax.axis_index("s")
    wid = cid * NS + sid

    z16 = jnp.zeros((16,), jnp.float32)
    for i in range(16):
        for c in range(D // 16):
            zbuf_v[i, pl.ds(c * 16, 16)] = z16

    base = sid * RPT

    def zacc(k, carry):
        pltpu.sync_copy(zbuf_v, acc_sh.at[pl.ds(base + k * 16, 16)])
        return carry

    lax.fori_loop(0, RPT // 16, zacc, 0)
    pltpu.sync_copy(zbuf_v.at[pl.ds(0, RPT - (RPT // 16) * 16)],
                    acc_sh.at[pl.ds(base + (RPT // 16) * 16,
                                    RPT - (RPT // 16) * 16)])

    pltpu.sync_copy(src_hbm.at[pl.ds(wid * RW, RW)], sidx_v)
    pltpu.sync_copy(dst_hbm.at[pl.ds(wid * RW, RW)], didx_v)
    plsc.subcore_barrier()

    def ebody(j, carry):
        pltpu.async_copy(g_hbm.at[sidx_v.at[j]], rows_v, sem).wait()
        pltpu.sync_copy(rows_v, acc_sh.at[didx_v.at[j]], add=True)
        return carry

    lax.fori_loop(0, RW, ebody, 0)
    plsc.subcore_barrier()
    pltpu.sync_copy(acc_sh.at[pl.ds(base, RPT)],
                    out_hbm.at[cid, pl.ds(base, RPT)])


def _dinv_body(degp_ref, dinv_ref):
    deg = degp_ref[0, :, 0] + degp_ref[1, :, 0] + 1.0
    dinv_ref[...] = lax.rsqrt(deg)[:, None]


def _pre_body(x_ref, w_ref, dinv_ref, g_ref):
    g_ref[...] = (x_ref[...] @ w_ref[...]) * dinv_ref[...]


def _mid_body(p_ref, g_ref, dinv_ref, bc_ref, w1_ref, b1_ref, w2_ref,
              b2_ref, wcn_ref, gn_ref):
    dinv = dinv_ref[...]
    z = dinv * (p_ref[0] + p_ref[1] + g_ref[...]) + bc_ref[...]
    t = jnp.maximum(z @ w1_ref[...] + b1_ref[...], 0.0)
    h = jnp.maximum(t @ w2_ref[...] + b2_ref[...], 0.0)
    gn_ref[...] = (h @ wcn_ref[...]) * dinv


def _fin_body(p_ref, g_ref, dinv_ref, bc_ref, w1_ref, b1_ref, w2_ref,
              b2_ref, f1w_ref, f1b_ref, f2w_ref, f2b_ref, o_ref):
    dinv = dinv_ref[...]
    z = dinv * (p_ref[0] + p_ref[1] + g_ref[...]) + bc_ref[...]
    t = jnp.maximum(z @ w1_ref[...] + b1_ref[...], 0.0)
    h = jnp.maximum(t @ w2_ref[...] + b2_ref[...], 0.0)
    u = jnp.maximum(h @ f1w_ref[...] + f1b_ref[...], 0.0)
    v = u @ f2w_ref[...] + f2b_ref[...]
    o_ref[...] = 1.0 / (1.0 + jnp.exp(-v))


def _full(shape):
    return pl.BlockSpec(shape, lambda i: (0,) * len(shape))


_dinv_call = pl.pallas_call(
    _dinv_body,
    out_shape=jax.ShapeDtypeStruct((NP, 1), jnp.float32),
)

_pre_call = pl.pallas_call(
    _pre_body,
    grid=(N // BLK,),
    in_specs=[
        pl.BlockSpec((BLK, D), lambda i: (i, 0)),
        _full((D, D)),
        pl.BlockSpec((BLK, 1), lambda i: (i, 0)),
    ],
    out_specs=pl.BlockSpec((BLK, D), lambda i: (i, 0)),
    out_shape=jax.ShapeDtypeStruct((N, D), jnp.float32),
)

_mid_call = pl.pallas_call(
    _mid_body,
    grid=(N // BLK,),
    in_specs=[
        pl.BlockSpec((NC, BLK, D), lambda i: (0, i, 0)),
        pl.BlockSpec((BLK, D), lambda i: (i, 0)),
        pl.BlockSpec((BLK, 1), lambda i: (i, 0)),
        _full((1, D)),
        _full((D, D)),
        _full((1, D)),
        _full((D, D)),
        _full((1, D)),
        _full((D, D)),
    ],
    out_specs=pl.BlockSpec((BLK, D), lambda i: (i, 0)),
    out_shape=jax.ShapeDtypeStruct((N, D), jnp.float32),
)

_fin_call = pl.pallas_call(
    _fin_body,
    grid=(N // BLK,),
    in_specs=[
        pl.BlockSpec((NC, BLK, D), lambda i: (0, i, 0)),
        pl.BlockSpec((BLK, D), lambda i: (i, 0)),
        pl.BlockSpec((BLK, 1), lambda i: (i, 0)),
        _full((1, D)),
        _full((D, D)),
        _full((1, D)),
        _full((D, D)),
        _full((1, D)),
        _full((D, H2)),
        _full((1, H2)),
        _full((H2, 1)),
        _full((1, 1)),
    ],
    out_specs=pl.BlockSpec((BLK, 1), lambda i: (i, 0)),
    out_shape=jax.ShapeDtypeStruct((N, 1), jnp.float32),
)


def kernel(x, edge_index, Wc0, bc0, m0w1, m0b1, m0w2, m0b2, Wc1, bc1,
           m1w1, m1b1, m1w2, m1b2, Wc2, bc2, m2w1, m2b1, m2w2, m2b2,
           fc1_w, fc1_b, fc2_w, fc2_b):
    src = edge_index[0]
    dst = edge_index[1]
    pad = E_PAD - E
    pi = jnp.arange(pad, dtype=jnp.int32)
    src_p = jnp.concatenate([src, (pi * 61) % N])
    dst_p = jnp.concatenate([dst, N + (pi % (NP - N))])
    src2 = src_p.reshape(ER, 128)
    dst2 = dst_p.reshape(ER, 128)

    degp = _deg_call(dst2)
    dinv_f = _dinv_call(degp)
    dinv = dinv_f[:N]

    bc0_ = bc0.reshape(1, D)
    bc1_ = bc1.reshape(1, D)
    bc2_ = bc2.reshape(1, D)

    g0 = _pre_call(x, Wc0, dinv)
    p0 = _scat_call(g0, src2, dst2)
    g1 = _mid_call(p0, g0, dinv, bc0_, m0w1, m0b1.reshape(1, D),
                   m0w2, m0b2.reshape(1, D), Wc1)
    p1 = _scat_call(g1, src2, dst2)
    g2 = _mid_call(p1, g1, dinv, bc1_, m1w1, m1b1.reshape(1, D),
                   m1w2, m1b2.reshape(1, D), Wc2)
    p2 = _scat_call(g2, src2, dst2)
    out = _fin_call(p2, g2, dinv, bc2_, m2w1, m2b1.reshape(1, D),
                    m2w2, m2b2.reshape(1, D),
                    fc1_w, fc1_b.reshape(1, H2), fc2_w, fc2_b.reshape(1, 1))
    return out


# speedup vs baseline: 21.8406x; 1.2423x over previous
import functools

import jax
import jax.numpy as jnp
from jax import lax
from jax.experimental import pallas as pl
from jax.experimental.pallas import tpu as pltpu
from jax.experimental.pallas import tpu_sc as plsc

N = 10000
D = 128
E = 320000
H2 = 64

NC = 2
NS = 16
NW = NC * NS

GW = 128
RW = 80
EW = RW * GW
E_PAD = NW * EW
ER = E_PAD // GW
NP = 10112
RPT = NP // NS

BLK = 400

_mesh = plsc.VectorSubcoreMesh(core_axis_name="c", subcore_axis_name="s")


DW = 16

@functools.partial(
    pl.kernel,
    out_type=jax.ShapeDtypeStruct((NC, NP, DW), jnp.float32),
    mesh=_mesh,
    scratch_types=[
        pltpu.VMEM((RW, GW), jnp.int32),
        pltpu.VMEM((GW, DW), jnp.float32),
        pltpu.VMEM((16, DW), jnp.float32),
        pltpu.VMEM_SHARED((NP, DW), jnp.float32),
    ],
)
def _deg_call(dst_hbm, out_hbm, didx_v, ones_v, zbuf_v, acc_sh):
    cid = lax.axis_index("c")
    sid = lax.axis_index("s")
    wid = cid * NS + sid

    one16 = jnp.ones((DW,), jnp.float32)
    z16 = jnp.zeros((DW,), jnp.float32)
    for i in range(16):
        zbuf_v[i, pl.ds(0, DW)] = z16
    for i in range(GW):
        ones_v[i, pl.ds(0, DW)] = one16

    base = sid * RPT

    def zacc(k, carry):
        pltpu.sync_copy(zbuf_v, acc_sh.at[pl.ds(base + k * 16, 16)])
        return carry

    lax.fori_loop(0, RPT // 16, zacc, 0)
    pltpu.sync_copy(zbuf_v.at[pl.ds(0, RPT - (RPT // 16) * 16)],
                    acc_sh.at[pl.ds(base + (RPT // 16) * 16,
                                    RPT - (RPT // 16) * 16)])
    pltpu.sync_copy(dst_hbm.at[pl.ds(wid * RW, RW)], didx_v)
    plsc.subcore_barrier()

    def body(j, carry):
        pltpu.sync_copy(ones_v, acc_sh.at[didx_v.at[j]], add=True)
        return carry

    lax.fori_loop(0, RW, body, 0)
    plsc.subcore_barrier()
    pltpu.sync_copy(acc_sh.at[pl.ds(base, RPT)],
                    out_hbm.at[cid, pl.ds(base, RPT)])


@functools.partial(
    pl.kernel,
    out_type=jax.ShapeDtypeStruct((NC, NP, D), jnp.float32),
    mesh=_mesh,
    scratch_types=[
        pltpu.VMEM((RW, GW), jnp.int32),
        pltpu.VMEM((2, GW), jnp.int32),
        pltpu.VMEM((2, GW), jnp.int32),
        pltpu.VMEM((2, GW, D), jnp.float32),
        pltpu.VMEM((16, D), jnp.float32),
        pltpu.VMEM_SHARED((NP, D), jnp.float32),
        pltpu.SemaphoreType.DMA,
        pltpu.SemaphoreType.DMA,
    ],
)
def _scat_call(g_hbm, pk_hbm, out_hbm,
               pk_v, sbuf_v, dbuf_v, rows_v, zbuf_v, acc_sh, sem0, sem1):
    sems = (sem0, sem1)
    cid = lax.axis_index("c")
    sid = lax.axis_index("s")
    wid = cid * NS + sid

    z16 = jnp.zeros((16,), jnp.float32)
    for i in range(16):
        for c in range(D // 16):
            zbuf_v[i, pl.ds(c * 16, 16)] = z16

    base = sid * RPT

    def zacc(k, carry):
        pltpu.sync_copy(zbuf_v, acc_sh.at[pl.ds(base + k * 16, 16)])
        return carry

    lax.fori_loop(0, RPT // 16, zacc, 0)
    pltpu.sync_copy(zbuf_v.at[pl.ds(0, RPT - (RPT // 16) * 16)],
                    acc_sh.at[pl.ds(base + (RPT // 16) * 16,
                                    RPT - (RPT // 16) * 16)])

    pltpu.sync_copy(pk_hbm.at[pl.ds(wid * RW, RW)], pk_v)
    plsc.subcore_barrier()

    m16 = jnp.full((16,), 0xFFFF, jnp.int32)
    s16 = jnp.full((16,), 16, jnp.int32)

    def unpack_group(r, b):
        for c in range(GW // 16):
            v = pk_v[r, pl.ds(c * 16, 16)]
            sbuf_v[b, pl.ds(c * 16, 16)] = v & m16
            dbuf_v[b, pl.ds(c * 16, 16)] = lax.shift_right_logical(v, s16)

    unpack_group(0, 0)
    pltpu.async_copy(g_hbm.at[sbuf_v.at[0]], rows_v.at[0], sem0).wait()

    def ebody(p, carry):
        for b in range(2):
            j = 2 * p + b
            jn_w = jnp.where(j + 1 < RW, j + 1, 0)
            unpack_group(jn_w, 1 - b)
            nxt = pltpu.async_copy(g_hbm.at[sbuf_v.at[1 - b]],
                                   rows_v.at[1 - b], sems[1 - b])
            pltpu.sync_copy(rows_v.at[b], acc_sh.at[dbuf_v.at[b]], add=True)
            nxt.wait()
        return carry

    lax.fori_loop(0, RW // 2, ebody, 0)
    plsc.subcore_barrier()
    pltpu.sync_copy(acc_sh.at[pl.ds(base, RPT)],
                    out_hbm.at[cid, pl.ds(base, RPT)])


def _dinv_body(degp_ref, dinv_ref):
    deg = degp_ref[0, :, 0] + degp_ref[1, :, 0] + 1.0
    dinv_ref[...] = lax.rsqrt(deg)[:, None]


def _pre_body(x_ref, w_ref, dinv_ref, g_ref):
    g_ref[...] = (x_ref[...] @ w_ref[...]) * dinv_ref[...]


def _mid_body(p_ref, g_ref, dinv_ref, bc_ref, w1_ref, b1_ref, w2_ref,
              b2_ref, wcn_ref, gn_ref):
    dinv = dinv_ref[...]
    z = dinv * (p_ref[0] + p_ref[1] + g_ref[...]) + bc_ref[...]
    t = jnp.maximum(z @ w1_ref[...] + b1_ref[...], 0.0)
    h = jnp.maximum(t @ w2_ref[...] + b2_ref[...], 0.0)
    gn_ref[...] = (h @ wcn_ref[...]) * dinv


def _fin_body(p_ref, g_ref, dinv_ref, bc_ref, w1_ref, b1_ref, w2_ref,
              b2_ref, f1w_ref, f1b_ref, f2w_ref, f2b_ref, o_ref):
    dinv = dinv_ref[...]
    z = dinv * (p_ref[0] + p_ref[1] + g_ref[...]) + bc_ref[...]
    t = jnp.maximum(z @ w1_ref[...] + b1_ref[...], 0.0)
    h = jnp.maximum(t @ w2_ref[...] + b2_ref[...], 0.0)
    u = jnp.maximum(h @ f1w_ref[...] + f1b_ref[...], 0.0)
    v = u @ f2w_ref[...] + f2b_ref[...]
    o_ref[...] = 1.0 / (1.0 + jnp.exp(-v))


def _full(shape):
    return pl.BlockSpec(shape, lambda i: (0,) * len(shape))


_dinv_call = pl.pallas_call(
    _dinv_body,
    out_shape=jax.ShapeDtypeStruct((NP, 1), jnp.float32),
)

_pre_call = pl.pallas_call(
    _pre_body,
    grid=(N // BLK,),
    in_specs=[
        pl.BlockSpec((BLK, D), lambda i: (i, 0)),
        _full((D, D)),
        pl.BlockSpec((BLK, 1), lambda i: (i, 0)),
    ],
    out_specs=pl.BlockSpec((BLK, D), lambda i: (i, 0)),
    out_shape=jax.ShapeDtypeStruct((N, D), jnp.float32),
)

_mid_call = pl.pallas_call(
    _mid_body,
    grid=(N // BLK,),
    in_specs=[
        pl.BlockSpec((NC, BLK, D), lambda i: (0, i, 0)),
        pl.BlockSpec((BLK, D), lambda i: (i, 0)),
        pl.BlockSpec((BLK, 1), lambda i: (i, 0)),
        _full((1, D)),
        _full((D, D)),
        _full((1, D)),
        _full((D, D)),
        _full((1, D)),
        _full((D, D)),
    ],
    out_specs=pl.BlockSpec((BLK, D), lambda i: (i, 0)),
    out_shape=jax.ShapeDtypeStruct((N, D), jnp.float32),
)

_fin_call = pl.pallas_call(
    _fin_body,
    grid=(N // BLK,),
    in_specs=[
        pl.BlockSpec((NC, BLK, D), lambda i: (0, i, 0)),
        pl.BlockSpec((BLK, D), lambda i: (i, 0)),
        pl.BlockSpec((BLK, 1), lambda i: (i, 0)),
        _full((1, D)),
        _full((D, D)),
        _full((1, D)),
        _full((D, D)),
        _full((1, D)),
        _full((D, H2)),
        _full((1, H2)),
        _full((H2, 1)),
        _full((1, 1)),
    ],
    out_specs=pl.BlockSpec((BLK, 1), lambda i: (i, 0)),
    out_shape=jax.ShapeDtypeStruct((N, 1), jnp.float32),
)


def kernel(x, edge_index, Wc0, bc0, m0w1, m0b1, m0w2, m0b2, Wc1, bc1,
           m1w1, m1b1, m1w2, m1b2, Wc2, bc2, m2w1, m2b1, m2w2, m2b2,
           fc1_w, fc1_b, fc2_w, fc2_b):
    src = edge_index[0]
    dst = edge_index[1]
    pad = E_PAD - E
    pi = jnp.arange(pad, dtype=jnp.int32)
    src_p = jnp.concatenate([src, (pi * 61) % N])
    dst_p = jnp.concatenate([dst, N + (pi % (NP - N))])
    dst2 = dst_p.reshape(ER, GW)
    pk2 = (src_p | (dst_p << 16)).reshape(ER, GW)

    degp = _deg_call(dst2)
    dinv_f = _dinv_call(degp)
    dinv = dinv_f[:N]

    bc0_ = bc0.reshape(1, D)
    bc1_ = bc1.reshape(1, D)
    bc2_ = bc2.reshape(1, D)

    g0 = _pre_call(x, Wc0, dinv)
    p0 = _scat_call(g0, pk2)
    g1 = _mid_call(p0, g0, dinv, bc0_, m0w1, m0b1.reshape(1, D),
                   m0w2, m0b2.reshape(1, D), Wc1)
    p1 = _scat_call(g1, pk2)
    g2 = _mid_call(p1, g1, dinv, bc1_, m1w1, m1b1.reshape(1, D),
                   m1w2, m1b2.reshape(1, D), Wc2)
    p2 = _scat_call(g2, pk2)
    out = _fin_call(p2, g2, dinv, bc2_, m2w1, m2b1.reshape(1, D),
                    m2w2, m2b2.reshape(1, D),
                    fc1_w, fc1_b.reshape(1, H2), fc2_w, fc2_b.reshape(1, 1))
    return out
